# baseline (speedup 1.0000x reference)
import functools
import math

import jax
import jax.numpy as jnp
from jax import lax
from jax.experimental import pallas as pl
from jax.experimental.pallas import tpu as pltpu
from jax.experimental.pallas import tpu_sc as plsc

NUM_HEADS = 16
NUM_BUCKETS = 32
MAX_DISTANCE = 128
Q = 2048
K = 2048
WPAD = 4352
B_PER_W = WPAD // 32
LAG = 128


def _bucket_kernel(out_ref):
    delta = jax.lax.broadcasted_iota(jnp.int32, (1, WPAD), 1) - (Q - 1)
    half = NUM_BUCKETS // 2
    rel_buckets = (delta > 0).astype(jnp.int32) * half
    a = jnp.abs(delta)
    max_exact = half // 2
    is_small = a < max_exact
    rel_large = max_exact + (
        jnp.log(a.astype(jnp.float32) / max_exact)
        / math.log(MAX_DISTANCE / max_exact)
        * (half - max_exact)
    ).astype(jnp.int32)
    rel_large = jnp.minimum(rel_large, half - 1)
    out_ref[...] = rel_buckets + jnp.where(is_small, a, rel_large)


def _sc_gather(table_hbm, idx_hbm, out_hbm, idx_v, rows_v, sem):
    wid = lax.axis_index("s") * 2 + lax.axis_index("c")
    base = wid * B_PER_W
    pltpu.sync_copy(idx_hbm.at[pl.ds(base, B_PER_W)], idx_v)
    pltpu.async_copy(table_hbm.at[idx_v], rows_v, sem).wait()
    pltpu.sync_copy(rows_v, out_hbm.at[pl.ds(base, B_PER_W)])


def _expand_kernel(tpad_ref, out_ref, tt_ref, sem):
    pending = []
    for m in range(16):
        for k in range(8):
            s = 8 * m + 7 - k
            tt_ref[:, m, k, :] = tpad_ref[:, s : s + 4096]
        for c in range(16):
            i = (Q - 8) - (128 * c + 8 * m)
            cp = pltpu.make_async_copy(
                tt_ref.at[:, m, :, pl.ds(128 * c, K)],
                out_ref.at[0, :, pl.ds(i, 8), :],
                sem,
            )
            cp.start()
            pending.append(cp)
            if len(pending) > LAG:
                pending.pop(0).wait()
    for cp in pending:
        cp.wait()


def kernel(bias_table, query_length, key_length):
    del query_length, key_length

    buckets = pl.pallas_call(
        _bucket_kernel,
        out_shape=jax.ShapeDtypeStruct((1, WPAD), jnp.int32),
    )()
    idx = buckets.reshape(WPAD)

    mesh = plsc.VectorSubcoreMesh(core_axis_name="c", subcore_axis_name="s")
    sc_gather = functools.partial(
        pl.kernel,
        mesh=mesh,
        out_type=jax.ShapeDtypeStruct((WPAD, 128), jnp.float32),
        scratch_types=[
            pltpu.VMEM((B_PER_W,), jnp.int32),
            pltpu.VMEM((B_PER_W, 128), jnp.float32),
            pltpu.SemaphoreType.DMA,
        ],
    )(_sc_gather)
    table_pad = jnp.pad(bias_table, ((0, 0), (0, 128 - NUM_HEADS)))
    tpad_t = sc_gather(table_pad, idx)
    tpad = tpad_t[:, :NUM_HEADS].T

    return pl.pallas_call(
        _expand_kernel,
        in_specs=[pl.BlockSpec(memory_space=pltpu.VMEM)],
        out_specs=pl.BlockSpec(memory_space=pl.ANY),
        out_shape=jax.ShapeDtypeStruct((1, NUM_HEADS, Q, K), jnp.float32),
        scratch_shapes=[
            pltpu.VMEM((NUM_HEADS, 16, 8, 4096), jnp.float32),
            pltpu.SemaphoreType.DMA,
        ],
    )(tpad)


# speedup vs baseline: 2.3445x; 2.3445x over previous
import math

import jax
import jax.numpy as jnp
from jax.experimental import pallas as pl
from jax.experimental.pallas import tpu as pltpu

NUM_HEADS = 16
NUM_BUCKETS = 32
MAX_DISTANCE = 128
Q = 2048
K = 2048
WPAD = 4224
LAG = 128


def _expand_kernel(tab_ref, out_ref, tpad_ref, tt_ref, sem):
    delta = jax.lax.broadcasted_iota(jnp.int32, (1, WPAD), 1) - (Q - 1)
    half = NUM_BUCKETS // 2
    rel_buckets = (delta > 0).astype(jnp.int32) * half
    a = jnp.abs(delta)
    max_exact = half // 2
    is_small = a < max_exact
    rel_large = max_exact + (
        jnp.log(a.astype(jnp.float32) / max_exact)
        / math.log(MAX_DISTANCE / max_exact)
        * (half - max_exact)
    ).astype(jnp.int32)
    rel_large = jnp.minimum(rel_large, half - 1)
    bucket = rel_buckets + jnp.where(is_small, a, rel_large)
    acc = jnp.zeros((NUM_HEADS, WPAD), dtype=jnp.float32)
    for b in range(NUM_BUCKETS):
        acc = jnp.where(bucket == b, tab_ref[:, b : b + 1], acc)
    tpad_ref[...] = acc

    pending = []
    for m in range(16):
        for k in range(8):
            s = 8 * m + 7 - k
            tt_ref[:, m, k, :] = tpad_ref[:, s : s + 4096]
        for c in range(16):
            i = (Q - 8) - (128 * c + 8 * m)
            cp = pltpu.make_async_copy(
                tt_ref.at[:, m, :, pl.ds(128 * c, K)],
                out_ref.at[0, :, pl.ds(i, 8), :],
                sem,
            )
            cp.start()
            pending.append(cp)
            if len(pending) > LAG:
                pending.pop(0).wait()
    for cp in pending:
        cp.wait()


def kernel(bias_table, query_length, key_length):
    del query_length, key_length
    tab_t = bias_table.T
    return pl.pallas_call(
        _expand_kernel,
        in_specs=[pl.BlockSpec(memory_space=pltpu.VMEM)],
        out_specs=pl.BlockSpec(memory_space=pl.ANY),
        out_shape=jax.ShapeDtypeStruct((1, NUM_HEADS, Q, K), jnp.float32),
        scratch_shapes=[
            pltpu.VMEM((NUM_HEADS, WPAD), jnp.float32),
            pltpu.VMEM((NUM_HEADS, 16, 8, 4096), jnp.float32),
            pltpu.SemaphoreType.DMA,
        ],
    )(tab_t)
